# Initial kernel scaffold; baseline (speedup 1.0000x reference)
import jax
import jax.numpy as jnp
from jax.experimental import pallas as pl


def kernel(q, k, v, attention_mask, k_top):
    raise NotImplementedError("write your pallas kernel here")


# speedup vs baseline: 12.1669x; 12.1669x over previous
import jax
import jax.numpy as jnp
from jax.experimental import pallas as pl
from jax.experimental.pallas import tpu as pltpu

_B, _H, _S, _D = 1, 12, 2048, 64
_BQ = 256


def _attn_kernel(ktop_ref, q_ref, k_ref, v_ref, bias_ref,
                 ctx_ref, probs_ref, mask_ref, est_ref):
    kt = ktop_ref[0]
    scale = 1.0 / jnp.sqrt(jnp.float32(_D))
    q = q_ref[0]
    kmat = k_ref[0]
    v = v_ref[0]
    s = jax.lax.dot_general(q, kmat, (((1,), (1,)), ((), ())),
                            preferred_element_type=jnp.float32)
    s = s * scale + bias_ref[0]

    m = jnp.max(s, axis=-1, keepdims=True)
    e = jnp.exp(s - m)
    denom = jnp.sum(e, axis=-1, keepdims=True)
    est = e / denom
    est_ref[0] = est

    u = jax.lax.bitcast_convert_type(est, jnp.int32)
    t = jnp.zeros((_BQ, 1), jnp.int32)
    for b in range(29, -1, -1):
        cand = t | (1 << b)
        cnt = jnp.sum((u >= cand).astype(jnp.int32), axis=-1, keepdims=True)
        t = jnp.where(cnt >= kt, cand, t)

    pm = u >= t
    mask_ref[0] = pm.astype(jnp.float32)
    ep = jnp.where(pm, e, 0.0)
    psum = jnp.sum(ep, axis=-1, keepdims=True)
    probs = ep / psum
    probs_ref[0] = probs
    ctx_ref[0] = jax.lax.dot_general(probs, v, (((1,), (0,)), ((), ())),
                                     preferred_element_type=jnp.float32)


def kernel(q, k, v, attention_mask, k_top):
    b, h, s, d = q.shape
    q3 = q.reshape(h, s, d)
    k3 = k.reshape(h, s, d)
    v3 = v.reshape(h, s, d)
    bias = attention_mask.reshape(1, s)
    kt = jnp.reshape(jnp.asarray(k_top, jnp.int32), (1,))

    grid = (h, s // _BQ)
    ctx, probs, pmask, est = pl.pallas_call(
        _attn_kernel,
        grid_spec=pltpu.PrefetchScalarGridSpec(
            num_scalar_prefetch=1,
            grid=grid,
            in_specs=[
                pl.BlockSpec((1, _BQ, d), lambda hh, qb, *_: (hh, qb, 0)),
                pl.BlockSpec((1, s, d), lambda hh, qb, *_: (hh, 0, 0)),
                pl.BlockSpec((1, s, d), lambda hh, qb, *_: (hh, 0, 0)),
                pl.BlockSpec((1, s), lambda hh, qb, *_: (0, 0)),
            ],
            out_specs=[
                pl.BlockSpec((1, _BQ, d), lambda hh, qb, *_: (hh, qb, 0)),
                pl.BlockSpec((1, _BQ, s), lambda hh, qb, *_: (hh, qb, 0)),
                pl.BlockSpec((1, _BQ, s), lambda hh, qb, *_: (hh, qb, 0)),
                pl.BlockSpec((1, _BQ, s), lambda hh, qb, *_: (hh, qb, 0)),
            ],
        ),
        out_shape=[
            jax.ShapeDtypeStruct((h, s, d), jnp.float32),
            jax.ShapeDtypeStruct((h, s, s), jnp.float32),
            jax.ShapeDtypeStruct((h, s, s), jnp.float32),
            jax.ShapeDtypeStruct((h, s, s), jnp.float32),
        ],
    )(kt, q3, k3, v3, bias)

    return (ctx.reshape(b, h, s, d),
            probs.reshape(b, h, s, s),
            pmask.reshape(b, h, s, s),
            est.reshape(b, h, s, s))


# speedup vs baseline: 12.1702x; 1.0003x over previous
import jax
import jax.numpy as jnp
from jax.experimental import pallas as pl
from jax.experimental.pallas import tpu as pltpu

_B, _H, _S, _D = 1, 12, 2048, 64
_BQ = 256


def _attn_kernel(ktop_ref, q_ref, k_ref, v_ref, bias_ref,
                 ctx_ref, probs_ref, mask_ref, est_ref):
    kt = ktop_ref[0]
    scale = 1.0 / jnp.sqrt(jnp.float32(_D))
    q = q_ref[0]
    kmat = k_ref[0]
    v = v_ref[0]
    s = jax.lax.dot_general(q, kmat, (((1,), (1,)), ((), ())),
                            preferred_element_type=jnp.float32)
    s = s * scale + bias_ref[0]

    m = jnp.max(s, axis=-1, keepdims=True)
    e = jnp.exp(s - m)
    denom = jnp.sum(e, axis=-1, keepdims=True)
    est = e / denom
    est_ref[0] = est

    u = jax.lax.bitcast_convert_type(est, jnp.int32)
    t = jnp.zeros((_BQ, 1), jnp.int32)
    for b in range(29, -1, -1):
        cand = t | (1 << b)
        cnt = jnp.sum(u >= cand, axis=-1, keepdims=True)
        t = jnp.where(cnt >= kt, cand, t)

    pm = u >= t
    mask_ref[0] = pm.astype(jnp.float32)
    ep = jnp.where(pm, e, 0.0)
    psum = jnp.sum(ep, axis=-1, keepdims=True)
    probs = ep / psum
    probs_ref[0] = probs
    ctx_ref[0] = jax.lax.dot_general(probs, v, (((1,), (0,)), ((), ())),
                                     preferred_element_type=jnp.float32)


def kernel(q, k, v, attention_mask, k_top):
    b, h, s, d = q.shape
    q3 = q.reshape(h, s, d)
    k3 = k.reshape(h, s, d)
    v3 = v.reshape(h, s, d)
    bias = attention_mask.reshape(1, s)
    kt = jnp.reshape(jnp.asarray(k_top, jnp.int32), (1,))

    grid = (h, s // _BQ)
    ctx, probs, pmask, est = pl.pallas_call(
        _attn_kernel,
        grid_spec=pltpu.PrefetchScalarGridSpec(
            num_scalar_prefetch=1,
            grid=grid,
            in_specs=[
                pl.BlockSpec((1, _BQ, d), lambda hh, qb, *_: (hh, qb, 0)),
                pl.BlockSpec((1, s, d), lambda hh, qb, *_: (hh, 0, 0)),
                pl.BlockSpec((1, s, d), lambda hh, qb, *_: (hh, 0, 0)),
                pl.BlockSpec((1, s), lambda hh, qb, *_: (0, 0)),
            ],
            out_specs=[
                pl.BlockSpec((1, _BQ, d), lambda hh, qb, *_: (hh, qb, 0)),
                pl.BlockSpec((1, _BQ, s), lambda hh, qb, *_: (hh, qb, 0)),
                pl.BlockSpec((1, _BQ, s), lambda hh, qb, *_: (hh, qb, 0)),
                pl.BlockSpec((1, _BQ, s), lambda hh, qb, *_: (hh, qb, 0)),
            ],
        ),
        out_shape=[
            jax.ShapeDtypeStruct((h, s, d), jnp.float32),
            jax.ShapeDtypeStruct((h, s, s), jnp.float32),
            jax.ShapeDtypeStruct((h, s, s), jnp.float32),
            jax.ShapeDtypeStruct((h, s, s), jnp.float32),
        ],
    )(kt, q3, k3, v3, bias)

    return (ctx.reshape(b, h, s, d),
            probs.reshape(b, h, s, s),
            pmask.reshape(b, h, s, s),
            est.reshape(b, h, s, s))


# speedup vs baseline: 16.4426x; 1.3511x over previous
import jax
import jax.numpy as jnp
from jax.experimental import pallas as pl
from jax.experimental.pallas import tpu as pltpu

_B, _H, _S, _D = 1, 12, 2048, 64
_BQ = 256


def _count_ge(x_b, cand_b, one_b, zero_b):
    ind = jnp.where(x_b >= cand_b, one_b, zero_b)
    n = ind.shape[-1]
    while n > 128:
        h = n // 2
        ind = ind[:, :h] + ind[:, h:]
        n = h
    return jnp.sum(ind.astype(jnp.float32), axis=-1, keepdims=True)


def _attn_kernel(ktop_ref, q_ref, k_ref, v_ref, bias_ref,
                 ctx_ref, probs_ref, mask_ref, est_ref):
    ktf = ktop_ref[0].astype(jnp.float32)
    scale = 1.0 / jnp.sqrt(jnp.float32(_D))
    q = q_ref[0]
    kmat = k_ref[0]
    v = v_ref[0]
    s = jax.lax.dot_general(q, kmat, (((1,), (1,)), ((), ())),
                            preferred_element_type=jnp.float32)
    s = s * scale + bias_ref[0]

    m = jnp.max(s, axis=-1, keepdims=True)
    e = jnp.exp(s - m)
    denom = jnp.sum(e, axis=-1, keepdims=True)
    est = e / denom
    est_ref[0] = est

    u = jax.lax.bitcast_convert_type(est, jnp.int32)
    bf = jnp.bfloat16
    one_b = jnp.array(1, bf)
    zero_b = jnp.array(0, bf)
    neg1_b = jnp.array(-1, bf)

    hi_b = jax.lax.bitcast_convert_type(u & jnp.int32(-65536),
                                        jnp.float32).astype(bf)
    mid_b = ((u >> 8) & 0xFF).astype(bf)
    lo_b = (u & 0xFF).astype(bf)

    def hi_pat(p):
        return jax.lax.bitcast_convert_type(p << 16, jnp.float32).astype(bf)

    p_i = jnp.zeros((_BQ, 1), jnp.int32)
    for b in range(13, -1, -1):
        cand_i = p_i | (1 << b)
        cnt = _count_ge(hi_b, hi_pat(cand_i), one_b, zero_b)
        p_i = jnp.where(cnt >= ktf, cand_i, p_i)
    c_gt = _count_ge(hi_b, hi_pat(p_i + 1), one_b, zero_b)

    midm = jnp.where(hi_b == hi_pat(p_i), mid_b, neg1_b)
    q_i = jnp.zeros((_BQ, 1), jnp.int32)
    for b in range(7, -1, -1):
        cand_i = q_i | (1 << b)
        cnt = c_gt + _count_ge(midm, cand_i.astype(bf), one_b, zero_b)
        q_i = jnp.where(cnt >= ktf, cand_i, q_i)
    c_gt2 = c_gt + _count_ge(midm, (q_i + 1).astype(bf), one_b, zero_b)

    lom = jnp.where(midm == q_i.astype(bf), lo_b, neg1_b)
    l_i = jnp.zeros((_BQ, 1), jnp.int32)
    for b in range(7, -1, -1):
        cand_i = l_i | (1 << b)
        cnt = c_gt2 + _count_ge(lom, cand_i.astype(bf), one_b, zero_b)
        l_i = jnp.where(cnt >= ktf, cand_i, l_i)

    t = (p_i << 16) | (q_i << 8) | l_i
    pm = u >= t
    mask_ref[0] = pm.astype(jnp.float32)
    ep = jnp.where(pm, e, 0.0)
    psum = jnp.sum(ep, axis=-1, keepdims=True)
    probs = ep / psum
    probs_ref[0] = probs
    ctx_ref[0] = jax.lax.dot_general(probs, v, (((1,), (0,)), ((), ())),
                                     preferred_element_type=jnp.float32)


def kernel(q, k, v, attention_mask, k_top):
    b, h, s, d = q.shape
    q3 = q.reshape(h, s, d)
    k3 = k.reshape(h, s, d)
    v3 = v.reshape(h, s, d)
    bias = attention_mask.reshape(1, s)
    kt = jnp.reshape(jnp.asarray(k_top, jnp.int32), (1,))

    grid = (h, s // _BQ)
    ctx, probs, pmask, est = pl.pallas_call(
        _attn_kernel,
        grid_spec=pltpu.PrefetchScalarGridSpec(
            num_scalar_prefetch=1,
            grid=grid,
            in_specs=[
                pl.BlockSpec((1, _BQ, d), lambda hh, qb, *_: (hh, qb, 0)),
                pl.BlockSpec((1, s, d), lambda hh, qb, *_: (hh, 0, 0)),
                pl.BlockSpec((1, s, d), lambda hh, qb, *_: (hh, 0, 0)),
                pl.BlockSpec((1, s), lambda hh, qb, *_: (0, 0)),
            ],
            out_specs=[
                pl.BlockSpec((1, _BQ, d), lambda hh, qb, *_: (hh, qb, 0)),
                pl.BlockSpec((1, _BQ, s), lambda hh, qb, *_: (hh, qb, 0)),
                pl.BlockSpec((1, _BQ, s), lambda hh, qb, *_: (hh, qb, 0)),
                pl.BlockSpec((1, _BQ, s), lambda hh, qb, *_: (hh, qb, 0)),
            ],
        ),
        out_shape=[
            jax.ShapeDtypeStruct((h, s, d), jnp.float32),
            jax.ShapeDtypeStruct((h, s, s), jnp.float32),
            jax.ShapeDtypeStruct((h, s, s), jnp.float32),
            jax.ShapeDtypeStruct((h, s, s), jnp.float32),
        ],
    )(kt, q3, k3, v3, bias)

    return (ctx.reshape(b, h, s, d),
            probs.reshape(b, h, s, s),
            pmask.reshape(b, h, s, s),
            est.reshape(b, h, s, s))


# speedup vs baseline: 16.7034x; 1.0159x over previous
import jax
import jax.numpy as jnp
from jax.experimental import pallas as pl
from jax.experimental.pallas import tpu as pltpu

_B, _H, _S, _D = 1, 12, 2048, 64
_BQ = 256


def _count_ge(x_b, cand_b, one_b, zero_b):
    ind = jnp.where(x_b >= cand_b, one_b, zero_b)
    n = ind.shape[-1]
    while n > 128:
        h = n // 2
        ind = ind[:, :h] + ind[:, h:]
        n = h
    return jnp.sum(ind.astype(jnp.float32), axis=-1, keepdims=True)


def _attn_kernel(ktop_ref, q_ref, k_ref, v_ref, bias_ref,
                 ctx_ref, probs_ref, mask_ref, est_ref):
    ktf = ktop_ref[0].astype(jnp.float32)
    scale = 1.0 / jnp.sqrt(jnp.float32(_D))
    q = q_ref[0, 0]
    kmat = k_ref[0, 0]
    v = v_ref[0, 0]
    s = jax.lax.dot_general(q, kmat, (((1,), (1,)), ((), ())),
                            preferred_element_type=jnp.float32)
    s = s * scale + bias_ref[0, 0]

    m = jnp.max(s, axis=-1, keepdims=True)
    e = jnp.exp(s - m)
    denom = jnp.sum(e, axis=-1, keepdims=True)
    est = e / denom
    est_ref[0, 0] = est

    u = jax.lax.bitcast_convert_type(est, jnp.int32)
    bf = jnp.bfloat16
    one_b = jnp.array(1, bf)
    zero_b = jnp.array(0, bf)
    neg1_b = jnp.array(-1, bf)

    hi_b = jax.lax.bitcast_convert_type(u & jnp.int32(-65536),
                                        jnp.float32).astype(bf)
    mid_b = ((u >> 8) & 0xFF).astype(bf)
    lo_b = (u & 0xFF).astype(bf)

    def hi_pat(p):
        return jax.lax.bitcast_convert_type(p << 16, jnp.float32).astype(bf)

    p_i = jnp.zeros((_BQ, 1), jnp.int32)
    for b in range(13, -1, -1):
        cand_i = p_i | (1 << b)
        cnt = _count_ge(hi_b, hi_pat(cand_i), one_b, zero_b)
        p_i = jnp.where(cnt >= ktf, cand_i, p_i)
    c_gt = _count_ge(hi_b, hi_pat(p_i + 1), one_b, zero_b)

    midm = jnp.where(hi_b == hi_pat(p_i), mid_b, neg1_b)
    q_i = jnp.zeros((_BQ, 1), jnp.int32)
    for b in range(7, -1, -1):
        cand_i = q_i | (1 << b)
        cnt = c_gt + _count_ge(midm, cand_i.astype(bf), one_b, zero_b)
        q_i = jnp.where(cnt >= ktf, cand_i, q_i)
    c_gt2 = c_gt + _count_ge(midm, (q_i + 1).astype(bf), one_b, zero_b)

    lom = jnp.where(midm == q_i.astype(bf), lo_b, neg1_b)
    l_i = jnp.zeros((_BQ, 1), jnp.int32)
    for b in range(7, -1, -1):
        cand_i = l_i | (1 << b)
        cnt = c_gt2 + _count_ge(lom, cand_i.astype(bf), one_b, zero_b)
        l_i = jnp.where(cnt >= ktf, cand_i, l_i)

    t = (p_i << 16) | (q_i << 8) | l_i
    pm = u >= t
    mask_ref[0, 0] = pm.astype(jnp.float32)
    ep = jnp.where(pm, e, 0.0)
    psum = jnp.sum(ep, axis=-1, keepdims=True)
    probs = ep / psum
    probs_ref[0, 0] = probs
    ctx_ref[0, 0] = jax.lax.dot_general(probs, v, (((1,), (0,)), ((), ())),
                                        preferred_element_type=jnp.float32)


def kernel(q, k, v, attention_mask, k_top):
    b, h, s, d = q.shape
    kt = jnp.reshape(jnp.asarray(k_top, jnp.int32), (1,))

    grid = (h, s // _BQ)
    ctx, probs, pmask, est = pl.pallas_call(
        _attn_kernel,
        grid_spec=pltpu.PrefetchScalarGridSpec(
            num_scalar_prefetch=1,
            grid=grid,
            in_specs=[
                pl.BlockSpec((1, 1, _BQ, d), lambda hh, qb, *_: (0, hh, qb, 0)),
                pl.BlockSpec((1, 1, s, d), lambda hh, qb, *_: (0, hh, 0, 0)),
                pl.BlockSpec((1, 1, s, d), lambda hh, qb, *_: (0, hh, 0, 0)),
                pl.BlockSpec((1, 1, 1, s), lambda hh, qb, *_: (0, 0, 0, 0)),
            ],
            out_specs=[
                pl.BlockSpec((1, 1, _BQ, d), lambda hh, qb, *_: (0, hh, qb, 0)),
                pl.BlockSpec((1, 1, _BQ, s), lambda hh, qb, *_: (0, hh, qb, 0)),
                pl.BlockSpec((1, 1, _BQ, s), lambda hh, qb, *_: (0, hh, qb, 0)),
                pl.BlockSpec((1, 1, _BQ, s), lambda hh, qb, *_: (0, hh, qb, 0)),
            ],
        ),
        out_shape=[
            jax.ShapeDtypeStruct((b, h, s, d), jnp.float32),
            jax.ShapeDtypeStruct((b, h, s, s), jnp.float32),
            jax.ShapeDtypeStruct((b, h, s, s), jnp.float32),
            jax.ShapeDtypeStruct((b, h, s, s), jnp.float32),
        ],
    )(kt, q, k, v, attention_mask)

    return (ctx, probs, pmask, est)
